# Initial kernel scaffold; baseline (speedup 1.0000x reference)
import jax
import jax.numpy as jnp
from jax.experimental import pallas as pl


def kernel(triples, hashes_sorted):
    raise NotImplementedError("write your pallas kernel here")


# speedup vs baseline: 14.4523x; 14.4523x over previous
import jax
import jax.numpy as jnp
from jax import lax
from jax.experimental import pallas as pl
from jax.experimental.pallas import tpu as pltpu
from jax.experimental.pallas import tpu_sc as plsc

K2 = 16
K1 = 256
BQ = 512
DMA_CHUNK = 128
UNROLL = 4

_MINI32 = -(2**31)


def _le64(th, tl, qh, ql):
    return (th < qh) | ((th == qh) & (tl <= ql))


def kernel(triples, hashes_sorted):
    L = hashes_sorted.shape[0]
    nF = -(-L // K1)
    Lp = nF * K1
    nrow = Lp // K2
    fence_rounds = max(1, (nF - 1).bit_length())

    th = (hashes_sorted >> 32).astype(jnp.int32)
    tl = ((hashes_sorted & 0xFFFFFFFF) - (1 << 31)).astype(jnp.int32)
    padv = jnp.full((Lp - L,), jnp.int32(2**31 - 1))
    th = jnp.concatenate([th, padv])
    tl = jnp.concatenate([tl, padv])
    T = jnp.concatenate([th.reshape(nrow, K2), tl.reshape(nrow, K2)], axis=1)
    D = jnp.concatenate(
        [th[::K2].reshape(nF, 16), tl[::K2].reshape(nF, 16)], axis=1)
    fhi_host = th[::K1]
    flo_host = tl[::K1]

    q = triples.reshape(-1, 3).astype(jnp.int32)
    N = q.shape[0]
    info = plsc.get_sparse_core_info()
    NC, NS = info.num_cores, info.num_subcores
    NW = NC * NS
    C = -(-N // (NW * BQ)) * BQ
    Npad = NW * C
    nb = C // BQ
    qpad = jnp.zeros((Npad - N,), jnp.int32)
    qs = jnp.concatenate([q[:, 0], qpad])
    qr = jnp.concatenate([q[:, 1], qpad])
    qo = jnp.concatenate([q[:, 2], qpad])

    def body(qs_h, qr_h, qo_h, fhi_h, flo_h, d_h, t_h, out_h,
             fhi_v, flo_v, sbuf, rbuf, obuf, qhib, qlob, fpos, trow,
             outb, dbuf, tbuf, sem):
        wid = lax.axis_index("s") * NC + lax.axis_index("c")
        pltpu.sync_copy(fhi_h, fhi_v)
        pltpu.sync_copy(flo_h, flo_v)
        iota = lax.iota(jnp.int32, 16)

        def batch(b, carry):
            off = wid * jnp.int32(C) + b * jnp.int32(BQ)
            pltpu.sync_copy(qs_h.at[pl.ds(off, BQ)], sbuf)
            pltpu.sync_copy(qr_h.at[pl.ds(off, BQ)], rbuf)
            pltpu.sync_copy(qo_h.at[pl.ds(off, BQ)], obuf)

            def phase1(g, c):
                for u in range(UNROLL):
                    v = g * jnp.int32(UNROLL) + jnp.int32(u)
                    sl = pl.ds(v * 16, 16)
                    s, r, o = sbuf[sl], rbuf[sl], obuf[sl]
                    qh = (s << 10) | (r >> 11)
                    ql = ((r << 21) | o) ^ jnp.int32(_MINI32)
                    qhib[sl] = qh
                    qlob[sl] = ql
                    lo = jnp.zeros((16,), jnp.int32)
                    hi = jnp.full((16,), nF, jnp.int32)
                    for _ in range(fence_rounds):
                        mid = (lo + hi) >> 1
                        fh = plsc.load_gather(fhi_v, [mid])
                        fl = plsc.load_gather(flo_v, [mid])
                        le = _le64(fh, fl, qh, ql)
                        lo = jnp.where(le, mid, lo)
                        hi = jnp.where(le, hi, mid)
                    fpos[sl] = lo
                return c

            lax.fori_loop(jnp.int32(0), jnp.int32(BQ // 16 // UNROLL), phase1, jnp.int32(0))

            cps = [pltpu.async_copy(
                       d_h.at[fpos.at[pl.ds(k * DMA_CHUNK, DMA_CHUNK)]],
                       dbuf.at[pl.ds(k * DMA_CHUNK, DMA_CHUNK)], sem)
                   for k in range(BQ // DMA_CHUNK)]
            for cp in cps:
                cp.wait()

            def phase2(g, c):
                for u in range(UNROLL):
                    v = g * jnp.int32(UNROLL) + jnp.int32(u)
                    sl = pl.ds(v * 16, 16)
                    qh, ql, f = qhib[sl], qlob[sl], fpos[sl]
                    qrow = iota + jnp.int32(v * 16)
                    base = jnp.zeros((16,), jnp.int32)
                    for w in (8, 4, 2, 1):
                        mid = base + w
                        dh = plsc.load_gather(dbuf, [qrow, mid])
                        dl = plsc.load_gather(dbuf, [qrow, mid + 16])
                        le = _le64(dh, dl, qh, ql)
                        base = jnp.where(le, mid, base)
                    trow[sl] = (f << 4) + base
                return c

            lax.fori_loop(jnp.int32(0), jnp.int32(BQ // 16 // UNROLL), phase2, jnp.int32(0))

            cps = [pltpu.async_copy(
                       t_h.at[trow.at[pl.ds(k * DMA_CHUNK, DMA_CHUNK)]],
                       tbuf.at[pl.ds(k * DMA_CHUNK, DMA_CHUNK)], sem)
                   for k in range(BQ // DMA_CHUNK)]
            for cp in cps:
                cp.wait()

            def phase3(g, c):
                for u in range(UNROLL):
                    v = g * jnp.int32(UNROLL) + jnp.int32(u)
                    sl = pl.ds(v * 16, 16)
                    qh, ql = qhib[sl], qlob[sl]
                    qrow = iota + jnp.int32(v * 16)
                    base = jnp.zeros((16,), jnp.int32)
                    for w in (8, 4, 2, 1):
                        mid = base + w
                        xh = plsc.load_gather(tbuf, [qrow, mid])
                        xl = plsc.load_gather(tbuf, [qrow, mid + 16])
                        le = _le64(xh, xl, qh, ql)
                        base = jnp.where(le, mid, base)
                    eh = plsc.load_gather(tbuf, [qrow, base])
                    el = plsc.load_gather(tbuf, [qrow, base + 16])
                    eq = (eh == qh) & (el == ql)
                    outb[sl] = jnp.where(eq, jnp.int32(0), jnp.int32(1))
                return c

            lax.fori_loop(jnp.int32(0), jnp.int32(BQ // 16 // UNROLL), phase3, jnp.int32(0))

            pltpu.sync_copy(outb, out_h.at[pl.ds(off, BQ)])
            return carry

        lax.fori_loop(jnp.int32(0), jnp.int32(nb), batch, jnp.int32(0))

    mesh = plsc.VectorSubcoreMesh(core_axis_name="c", subcore_axis_name="s")
    run = pl.kernel(
        body,
        out_type=jax.ShapeDtypeStruct((Npad,), jnp.int32),
        mesh=mesh,
        compiler_params=pltpu.CompilerParams(
            needs_layout_passes=False, use_tc_tiling_on_sc=False),
        scratch_types=[
            pltpu.VMEM((nF,), jnp.int32),
            pltpu.VMEM((nF,), jnp.int32),
            pltpu.VMEM((BQ,), jnp.int32),
            pltpu.VMEM((BQ,), jnp.int32),
            pltpu.VMEM((BQ,), jnp.int32),
            pltpu.VMEM((BQ,), jnp.int32),
            pltpu.VMEM((BQ,), jnp.int32),
            pltpu.VMEM((BQ,), jnp.int32),
            pltpu.VMEM((BQ,), jnp.int32),
            pltpu.VMEM((BQ,), jnp.int32),
            pltpu.VMEM((BQ, 32), jnp.int32),
            pltpu.VMEM((BQ, 32), jnp.int32),
            pltpu.SemaphoreType.DMA,
        ],
    )
    out = run(qs, qr, qo, fhi_host, flo_host, D, T)
    return (out[:N] > 0).reshape(triples.shape[:-1])


# speedup vs baseline: 17.5941x; 1.2174x over previous
import jax
import jax.numpy as jnp
from jax import lax
from jax.experimental import pallas as pl
from jax.experimental.pallas import tpu as pltpu
from jax.experimental.pallas import tpu_sc as plsc

K2 = 16
K1 = 256
BQ = 512
CH = 128
UNROLL = 4
GROUPS = BQ // 16 // UNROLL

_MINI32 = -(2**31)


def _le64(th, tl, qh, ql):
    return (th < qh) | ((th == qh) & (tl <= ql))


def _hash16(sb, rb, ob, sl):
    s, r, o = sb[sl], rb[sl], ob[sl]
    qh = (s << 10) | (r >> 11)
    ql = ((r << 21) | o) ^ jnp.int32(_MINI32)
    return qh, ql


def _searchrow(buf, qrow, qh, ql):
    base = jnp.zeros((16,), jnp.int32)
    for w in (8, 4, 2, 1):
        mid = base + w
        xh = plsc.load_gather(buf, [qrow, mid])
        xl = plsc.load_gather(buf, [qrow, mid + 16])
        base = jnp.where(_le64(xh, xl, qh, ql), mid, base)
    return base


def kernel(triples, hashes_sorted):
    L = hashes_sorted.shape[0]
    nF = -(-L // K1)
    Lp = nF * K1
    nrow = Lp // K2
    fence_rounds = max(1, (nF - 1).bit_length())

    th = (hashes_sorted >> 32).astype(jnp.int32)
    tl = ((hashes_sorted & 0xFFFFFFFF) - (1 << 31)).astype(jnp.int32)
    padv = jnp.full((Lp - L,), jnp.int32(2**31 - 1))
    th = jnp.concatenate([th, padv])
    tl = jnp.concatenate([tl, padv])
    t_rows = jnp.concatenate(
        [th.reshape(nrow, K2), tl.reshape(nrow, K2)], axis=1)
    d_rows = jnp.concatenate(
        [th[::K2].reshape(nF, 16), tl[::K2].reshape(nF, 16)], axis=1)
    fhi_host = th[::K1]
    flo_host = tl[::K1]

    q = triples.reshape(-1, 3).astype(jnp.int32)
    N = q.shape[0]
    info = plsc.get_sparse_core_info()
    NC, NS = info.num_cores, info.num_subcores
    NW = NC * NS
    C = -(-N // (NW * 2 * BQ)) * (2 * BQ)
    nb = C // BQ
    Npad = NW * C
    zpad = jnp.zeros((Npad - N + 2 * BQ,), jnp.int32)
    qs = jnp.concatenate([q[:, 0], zpad])
    qr = jnp.concatenate([q[:, 1], zpad])
    qo = jnp.concatenate([q[:, 2], zpad])

    mesh = plsc.VectorSubcoreMesh(core_axis_name="c", subcore_axis_name="s")
    cparams = pltpu.CompilerParams(
        needs_layout_passes=False, use_tc_tiling_on_sc=False)
    i32 = jnp.int32

    def body1(qs_h, qr_h, qo_h, fhi_h, flo_h, fpos_h,
              fhi_v, flo_v, sb0, sb1, rb0, rb1, ob0, ob1, fb0, fb1,
              sin0, sin1, sout0, sout1):
        wid = lax.axis_index("s") * NC + lax.axis_index("c")
        base = wid * i32(C)
        pltpu.sync_copy(fhi_h, fhi_v)
        pltpu.sync_copy(flo_h, flo_v)
        sbs, rbs, obs, fbs = (sb0, sb1), (rb0, rb1), (ob0, ob1), (fb0, fb1)
        sins, souts = (sin0, sin1), (sout0, sout1)

        def fire_lin(i, p):
            off = base + i * i32(BQ)
            pltpu.async_copy(qs_h.at[pl.ds(off, BQ)], sbs[p], sins[p])
            pltpu.async_copy(qr_h.at[pl.ds(off, BQ)], rbs[p], sins[p])
            pltpu.async_copy(qo_h.at[pl.ds(off, BQ)], obs[p], sins[p])

        def wait_lin(p):
            for dst in (sbs[p], rbs[p], obs[p]):
                pltpu.make_async_copy(qs_h.at[pl.ds(0, BQ)], dst,
                                      sins[p]).wait()

        def fire_out(i, p):
            off = base + i * i32(BQ)
            pltpu.async_copy(fbs[p], fpos_h.at[pl.ds(off, BQ)], souts[p])

        def wait_out(p):
            pltpu.make_async_copy(fbs[p], fpos_h.at[pl.ds(0, BQ)],
                                  souts[p]).wait()

        fire_lin(i32(0), 0)
        fire_lin(i32(1), 1)
        fire_out(i32(0), 0)
        fire_out(i32(1), 1)

        def half(i, p):
            wait_lin(p)
            wait_out(p)

            def phase1(g, c):
                for u in range(UNROLL):
                    v = g * i32(UNROLL) + i32(u)
                    sl = pl.ds(v * 16, 16)
                    qh, ql = _hash16(sbs[p], rbs[p], obs[p], sl)
                    lo = jnp.zeros((16,), i32)
                    hi = jnp.full((16,), nF, i32)
                    for _ in range(fence_rounds):
                        mid = (lo + hi) >> 1
                        fh = plsc.load_gather(fhi_v, [mid])
                        fl = plsc.load_gather(flo_v, [mid])
                        le = _le64(fh, fl, qh, ql)
                        lo = jnp.where(le, mid, lo)
                        hi = jnp.where(le, hi, mid)
                    fbs[p][sl] = lo
                return c

            lax.fori_loop(i32(0), i32(GROUPS), phase1, i32(0))
            fire_out(i, p)
            fire_lin(i + i32(2), p)

        def pair(t, c):
            i = t * i32(2)
            half(i, 0)
            half(i + i32(1), 1)
            return c

        lax.fori_loop(i32(0), i32(nb // 2), pair, i32(0))
        for p in (0, 1):
            wait_out(p)
            wait_lin(p)

    p1 = pl.kernel(
        body1,
        out_type=jax.ShapeDtypeStruct((Npad,), i32),
        mesh=mesh,
        compiler_params=cparams,
        scratch_types=[
            pltpu.VMEM((nF,), i32), pltpu.VMEM((nF,), i32),
            pltpu.VMEM((BQ,), i32), pltpu.VMEM((BQ,), i32),
            pltpu.VMEM((BQ,), i32), pltpu.VMEM((BQ,), i32),
            pltpu.VMEM((BQ,), i32), pltpu.VMEM((BQ,), i32),
            pltpu.VMEM((BQ,), i32), pltpu.VMEM((BQ,), i32),
            pltpu.SemaphoreType.DMA, pltpu.SemaphoreType.DMA,
            pltpu.SemaphoreType.DMA, pltpu.SemaphoreType.DMA,
        ],
    )

    def make_gather_pass(rows_hbm_shape, compute):
        def body(qs_h, qr_h, qo_h, idx_h, rows_h, res_h,
                 sb0, sb1, rb0, rb1, ob0, ob1, ib0, ib1,
                 gb0, gb1, eb0, eb1,
                 sin0, sin1, sg0, sg1, sout0, sout1):
            wid = lax.axis_index("s") * NC + lax.axis_index("c")
            base = wid * i32(C)
            sbs, rbs, obs = (sb0, sb1), (rb0, rb1), (ob0, ob1)
            ibs, gbs, ebs = (ib0, ib1), (gb0, gb1), (eb0, eb1)
            sins, sgs, souts = (sin0, sin1), (sg0, sg1), (sout0, sout1)

            def fire_lin(i, p):
                off = base + i * i32(BQ)
                pltpu.async_copy(qs_h.at[pl.ds(off, BQ)], sbs[p], sins[p])
                pltpu.async_copy(qr_h.at[pl.ds(off, BQ)], rbs[p], sins[p])
                pltpu.async_copy(qo_h.at[pl.ds(off, BQ)], obs[p], sins[p])
                pltpu.async_copy(idx_h.at[pl.ds(off, BQ)], ibs[p], sins[p])

            def wait_lin(p):
                for dst in (sbs[p], rbs[p], obs[p], ibs[p]):
                    pltpu.make_async_copy(qs_h.at[pl.ds(0, BQ)], dst,
                                          sins[p]).wait()

            def fire_gather(p):
                for k in range(BQ // CH):
                    pltpu.async_copy(
                        rows_h.at[ibs[p].at[pl.ds(k * CH, CH)]],
                        gbs[p].at[pl.ds(k * CH, CH)], sgs[p])

            def wait_gather(p):
                for k in range(BQ // CH):
                    pltpu.make_async_copy(
                        rows_h.at[pl.ds(0, CH)],
                        gbs[p].at[pl.ds(k * CH, CH)], sgs[p]).wait()

            def fire_out(i, p):
                off = base + i * i32(BQ)
                pltpu.async_copy(ebs[p], res_h.at[pl.ds(off, BQ)], souts[p])

            def wait_out(p):
                pltpu.make_async_copy(ebs[p], res_h.at[pl.ds(0, BQ)],
                                      souts[p]).wait()

            fire_lin(i32(0), 0)
            fire_lin(i32(1), 1)
            wait_lin(0)
            fire_gather(0)
            fire_out(i32(0), 0)
            fire_out(i32(1), 1)

            def half(i, p):
                wait_lin(1 - p)
                fire_gather(1 - p)
                wait_gather(p)
                wait_out(p)
                compute(p, sbs, rbs, obs, ibs, gbs, ebs)
                fire_out(i, p)
                fire_lin(i + i32(2), p)

            def pair(t, c):
                i = t * i32(2)
                half(i, 0)
                half(i + i32(1), 1)
                return c

            lax.fori_loop(i32(0), i32(nb // 2), pair, i32(0))
            wait_gather(nb & 1)
            wait_lin((nb + 1) & 1)
            for p in (0, 1):
                wait_out(p)

        return pl.kernel(
            body,
            out_type=jax.ShapeDtypeStruct((Npad,), i32),
            mesh=mesh,
            compiler_params=cparams,
            scratch_types=[
                pltpu.VMEM((BQ,), i32), pltpu.VMEM((BQ,), i32),
                pltpu.VMEM((BQ,), i32), pltpu.VMEM((BQ,), i32),
                pltpu.VMEM((BQ,), i32), pltpu.VMEM((BQ,), i32),
                pltpu.VMEM((BQ,), i32), pltpu.VMEM((BQ,), i32),
                pltpu.VMEM((BQ, 32), i32), pltpu.VMEM((BQ, 32), i32),
                pltpu.VMEM((BQ,), i32), pltpu.VMEM((BQ,), i32),
                pltpu.SemaphoreType.DMA, pltpu.SemaphoreType.DMA,
                pltpu.SemaphoreType.DMA, pltpu.SemaphoreType.DMA,
                pltpu.SemaphoreType.DMA, pltpu.SemaphoreType.DMA,
            ],
        )

    def compute2(p, sbs, rbs, obs, ibs, gbs, ebs):
        it = lax.iota(i32, 16)

        def phase2(g, c):
            for u in range(UNROLL):
                v = g * i32(UNROLL) + i32(u)
                sl = pl.ds(v * 16, 16)
                qh, ql = _hash16(sbs[p], rbs[p], obs[p], sl)
                f = ibs[p][sl]
                qrow = it + v * i32(16)
                j = _searchrow(gbs[p], qrow, qh, ql)
                ebs[p][sl] = (f << 4) + j
            return c

        lax.fori_loop(i32(0), i32(GROUPS), phase2, i32(0))

    def compute3(p, sbs, rbs, obs, ibs, gbs, ebs):
        it = lax.iota(i32, 16)

        def phase3(g, c):
            for u in range(UNROLL):
                v = g * i32(UNROLL) + i32(u)
                sl = pl.ds(v * 16, 16)
                qh, ql = _hash16(sbs[p], rbs[p], obs[p], sl)
                qrow = it + v * i32(16)
                j = _searchrow(gbs[p], qrow, qh, ql)
                eh = plsc.load_gather(gbs[p], [qrow, j])
                el = plsc.load_gather(gbs[p], [qrow, j + 16])
                eq = (eh == qh) & (el == ql)
                ebs[p][sl] = jnp.where(eq, i32(0), i32(1))
            return c

        lax.fori_loop(i32(0), i32(GROUPS), phase3, i32(0))

    p2 = make_gather_pass((nF, 32), compute2)
    p3 = make_gather_pass((nrow, 32), compute3)

    ztail = jnp.zeros((2 * BQ,), i32)
    fpos = p1(qs, qr, qo, fhi_host, flo_host)
    fpos_p = jnp.concatenate([fpos, ztail])
    trow = p2(qs, qr, qo, fpos_p, d_rows)
    trow_p = jnp.concatenate([trow, ztail])
    res = p3(qs, qr, qo, trow_p, t_rows)
    return (res[:N] > 0).reshape(triples.shape[:-1])
